# baseline (speedup 1.0000x reference)
import functools

import jax
import jax.numpy as jnp
from jax import lax
from jax.experimental import pallas as pl
from jax.experimental.pallas import tpu as pltpu
from jax.experimental.pallas import tpu_sc as plsc

N = 10000
E = 160000
F = 128
FH = F // 2
NRELS = 4
NRULES = 3
KR = NRELS * NRULES
RW = 256

NC = 2
NS = 16
C = 32
NCHT = 5120
E_PAD = NCHT * C
NCH = NCHT // NS
MDW = 256
N_PAD = 10240
NAGG = N_PAD // 2
RPT = NAGG // NS


def _a_body(feat_ref, w0_ref, w1_ref, lw_ref, b_ref, t0_ref, t1_ref, sl_ref):
    x = feat_ref[...]
    for k in range(NRELS):
        t0_ref[:, k * RW:(k + 1) * RW] = jnp.dot(
            x, w0_ref[k], preferred_element_type=jnp.float32)
        t1_ref[:, k * RW:(k + 1) * RW] = jnp.dot(
            x, w1_ref[k], preferred_element_type=jnp.float32)
    sl_ref[...] = jnp.dot(x, lw_ref[...],
                          preferred_element_type=jnp.float32) + b_ref[...]


def _phase_a(feat, w0p, w1p, loop_weight, bias2d):
    br = 2000
    grid = N // br
    return pl.pallas_call(
        _a_body,
        grid=(grid,),
        in_specs=[
            pl.BlockSpec((br, F), lambda i: (i, 0)),
            pl.BlockSpec((NRELS, F, RW), lambda i: (0, 0, 0)),
            pl.BlockSpec((NRELS, F, RW), lambda i: (0, 0, 0)),
            pl.BlockSpec((F, F), lambda i: (0, 0)),
            pl.BlockSpec((1, F), lambda i: (0, 0)),
        ],
        out_specs=[
            pl.BlockSpec((br, NRELS * RW), lambda i: (i, 0)),
            pl.BlockSpec((br, NRELS * RW), lambda i: (i, 0)),
            pl.BlockSpec((br, F), lambda i: (i, 0)),
        ],
        out_shape=[
            jax.ShapeDtypeStruct((N, NRELS * RW), jnp.float32),
            jax.ShapeDtypeStruct((N, NRELS * RW), jnp.float32),
            jax.ShapeDtypeStruct((N, F), jnp.float32),
        ],
    )(feat, w0p, w1p, loop_weight, bias2d)


_SPLAT_DNUMS = lax.GatherDimensionNumbers(
    offset_dims=(), collapsed_slice_dims=(0,), start_index_map=(0,))


def _splat(v, j):
    idx = jnp.full((16, 1), j, jnp.int32)
    return lax.gather(v, idx, _SPLAT_DNUMS, slice_sizes=(1,),
                      mode=lax.GatherScatterMode.PROMISE_IN_BOUNDS)


def _b_body(t0_hbm, t1_hbm, md_hbm, zero_hbm, out_hbm,
            md_v, idx_v, dst_v, par_v, rows_v, msg_v, agg,
            sem_m, sem_g, sem_s):
    cid = lax.axis_index("c")
    sid = lax.axis_index("s")

    pltpu.sync_copy(zero_hbm.at[pl.ds(sid * RPT, RPT)],
                    agg.at[pl.ds(sid * RPT, RPT)])
    plsc.subcore_barrier()

    row0 = sid * NCH

    def start_md(g, s8):
        pltpu.async_copy(md_hbm.at[pl.ds((row0 + g) * MDW, MDW)],
                         md_v.at[s8], sem_m[s8])

    def wait_md(s8):
        pltpu.make_async_copy(md_hbm.at[pl.ds(0, MDW)], md_v.at[s8],
                              sem_m[s8]).wait()

    def prep_gather(s8, s4):
        for j in range(C // 16):
            sl = pl.ds(j * 16, 16)
            fidx = (md_v[s8, pl.ds(j * 16, 16)] * float(NRELS)
                    + md_v[s8, pl.ds(C + j * 16, 16)])
            idx_v[s4, sl] = fidx.astype(jnp.int32)
            di = md_v[s8, pl.ds(2 * C + j * 16, 16)].astype(jnp.int32)
            dst_v[s8, sl] = lax.shift_right_logical(di, 1)
            par_v[s8, sl] = lax.bitwise_and(di, 1).astype(jnp.float32)

        @pl.when(cid == 0)
        def _():
            pltpu.async_copy(t0_hbm.at[idx_v.at[s4]], rows_v.at[s4],
                             sem_g[s4])

        @pl.when(cid == 1)
        def _():
            pltpu.async_copy(t1_hbm.at[idx_v.at[s4]], rows_v.at[s4],
                             sem_g[s4])

    def wait_gather(s4):
        pltpu.make_async_copy(t0_hbm.at[idx_v.at[s4]], rows_v.at[s4],
                              sem_g[s4]).wait()

    def wait_scatter(b2):
        pltpu.make_async_copy(msg_v.at[b2], agg.at[dst_v.at[0]],
                              sem_s[b2]).wait()

    def compute_chunk(s8, s4, b2):
        tq0 = md_v[s8, pl.ds(3 * C, 16)]
        tq1 = md_v[s8, pl.ds(4 * C, 16)]
        tq2 = md_v[s8, pl.ds(5 * C, 16)]
        tr0 = md_v[s8, pl.ds(3 * C + 16, 16)]
        tr1 = md_v[s8, pl.ds(4 * C + 16, 16)]
        tr2 = md_v[s8, pl.ds(5 * C + 16, 16)]
        pq = par_v[s8, pl.ds(0, 16)]
        pr = par_v[s8, pl.ds(16, 16)]
        one = jnp.full((16,), 1.0, jnp.float32)

        def lane_body(j, c3):
            t0 = _splat(tq0, j)
            t1 = _splat(tq1, j)
            t2 = _splat(tq2, j)
            phi = _splat(pq, j)
            plo = one - phi
            for s in range(FH // 16):
                a = rows_v[s4, j, pl.ds(s * 16, 16)] * t0
                a = a + rows_v[s4, j, pl.ds(FH + s * 16, 16)] * t1
                a = a + rows_v[s4, j, pl.ds(2 * FH + s * 16, 16)] * t2
                msg_v[b2, j, pl.ds(s * 16, 16)] = a * plo
                msg_v[b2, j, pl.ds(FH + s * 16, 16)] = a * phi
            return c3

        def lane_body2(j, c3):
            jj = j + 16
            t0 = _splat(tr0, j)
            t1 = _splat(tr1, j)
            t2 = _splat(tr2, j)
            phi = _splat(pr, j)
            plo = one - phi
            for s in range(FH // 16):
                a = rows_v[s4, jj, pl.ds(s * 16, 16)] * t0
                a = a + rows_v[s4, jj, pl.ds(FH + s * 16, 16)] * t1
                a = a + rows_v[s4, jj, pl.ds(2 * FH + s * 16, 16)] * t2
                msg_v[b2, jj, pl.ds(s * 16, 16)] = a * plo
                msg_v[b2, jj, pl.ds(FH + s * 16, 16)] = a * phi
            return c3

        lax.fori_loop(0, 16, lane_body, 0)
        lax.fori_loop(0, 16, lane_body2, 0)
        pltpu.async_copy(msg_v.at[b2], agg.at[dst_v.at[s8]], sem_s[b2],
                         add=True)

    for g in range(8):
        start_md(g, g)
    for g in range(3):
        wait_md(g)
        prep_gather(g, g)

    def oct_body(i, carry):
        for b8 in range(8):
            g = i * 8 + b8
            b4 = b8 % 4
            b2 = b8 % 2

            @pl.when(g + 3 < NCH)
            def _():
                wait_md((b8 + 3) % 8)
                prep_gather((b8 + 3) % 8, (b4 + 3) % 4)

            wait_gather(b4)

            @pl.when(g >= 2)
            def _():
                wait_scatter(b2)

            compute_chunk(b8, b4, b2)

            @pl.when(g + 8 < NCH)
            def _():
                start_md(g + 8, b8)

        return carry

    lax.fori_loop(0, NCH // 8, oct_body, 0)
    wait_scatter(0)
    wait_scatter(1)

    plsc.subcore_barrier()
    pltpu.sync_copy(agg.at[pl.ds(sid * RPT, RPT)],
                    out_hbm.at[cid, pl.ds(sid * RPT, RPT)])


_phase_b = functools.partial(
    pl.kernel,
    out_type=jax.ShapeDtypeStruct((NC, NAGG, F), jnp.float32),
    mesh=plsc.VectorSubcoreMesh(core_axis_name="c", subcore_axis_name="s"),
    scratch_types=[
        pltpu.VMEM((8, MDW), jnp.float32),
        pltpu.VMEM((4, C), jnp.int32),
        pltpu.VMEM((8, C), jnp.int32),
        pltpu.VMEM((8, C), jnp.float32),
        pltpu.VMEM((4, C, RW), jnp.float32),
        pltpu.VMEM((2, C, F), jnp.float32),
        pltpu.VMEM_SHARED((NAGG, F), jnp.float32),
        [pltpu.SemaphoreType.DMA] * 8,
        [pltpu.SemaphoreType.DMA] * 4,
        [pltpu.SemaphoreType.DMA] * 2,
    ],
)(_b_body)


def _c_body(p_ref, sl_ref, out_ref):
    out_ref[:, :FH] = p_ref[0] + sl_ref[:, :FH]
    out_ref[:, FH:] = p_ref[1] + sl_ref[:, FH:]


def _phase_c(partials, selfloop):
    br = 1000
    grid = N // br
    return pl.pallas_call(
        _c_body,
        grid=(grid,),
        in_specs=[
            pl.BlockSpec((NC, br, FH), lambda i: (0, i, 0)),
            pl.BlockSpec((br, F), lambda i: (i, 0)),
        ],
        out_specs=pl.BlockSpec((br, F), lambda i: (i, 0)),
        out_shape=jax.ShapeDtypeStruct((N, F), jnp.float32),
    )(partials, selfloop)


def kernel(feat, edge_index, etypes, truth_value, loop_weight, weight, h_bias):
    wt = weight.transpose(0, 2, 1, 3)
    w0p = jnp.pad(wt[..., :FH].reshape(NRELS, F, NRULES * FH),
                  ((0, 0), (0, 0), (0, RW - NRULES * FH)))
    w1p = jnp.pad(wt[..., FH:].reshape(NRELS, F, NRULES * FH),
                  ((0, 0), (0, 0), (0, RW - NRULES * FH)))
    bias2d = h_bias.reshape(1, F)
    t0, t1, selfloop = _phase_a(feat, w0p, w1p, loop_weight, bias2d)
    t0 = t0.reshape(N * NRELS, RW)
    t1 = t1.reshape(N * NRELS, RW)

    pad = E_PAD - E
    src = jnp.concatenate(
        [edge_index[0], jnp.zeros((pad,), jnp.int32)]).reshape(NCHT, C)
    et = jnp.concatenate(
        [etypes, jnp.zeros((pad,), jnp.int32)]).reshape(NCHT, C)
    dst = jnp.concatenate(
        [edge_index[1], jnp.zeros((pad,), jnp.int32)]).reshape(NCHT, C)
    tru = jnp.concatenate(
        [truth_value.reshape(E, NRULES),
         jnp.zeros((pad, NRULES), jnp.float32)])
    tru = tru.reshape(NCHT, C, NRULES).transpose(0, 2, 1).reshape(NCHT, 3 * C)
    md = jnp.concatenate(
        [src.astype(jnp.float32), et.astype(jnp.float32),
         dst.astype(jnp.float32), tru,
         jnp.zeros((NCHT, MDW - 6 * C), jnp.float32)], axis=1).reshape(-1)
    zero = jnp.zeros((NAGG, F), jnp.float32)

    partials = _phase_b(t0, t1, md, zero)
    return _phase_c(partials.reshape(NC, N_PAD, FH), selfloop)


# speedup vs baseline: 1.1814x; 1.1814x over previous
import functools

import jax
import jax.numpy as jnp
from jax import lax
from jax.experimental import pallas as pl
from jax.experimental.pallas import tpu as pltpu
from jax.experimental.pallas import tpu_sc as plsc

N = 10000
E = 160000
F = 128
NRELS = 4
NRULES = 3
KR = NRELS * NRULES
RW = NRULES * F

NC = 2
NS = 16
NW = NC * NS
C = 16
NCHT = 10240
E_PAD = NCHT * C
NCH = NCHT // NW
MDW = 3 * C
TW = NRULES * C
N_PAD = 10112
RPT = N_PAD // NS


def _a_body(feat_ref, w_ref, lw_ref, b_ref, g_ref, sl_ref):
    x = feat_ref[...]
    xb = x.astype(jnp.bfloat16)
    for j in range(KR):
        g_ref[:, j * F:(j + 1) * F] = jnp.dot(
            xb, w_ref[j].astype(jnp.bfloat16),
            preferred_element_type=jnp.float32)
    sl_ref[...] = jnp.dot(x, lw_ref[...],
                          preferred_element_type=jnp.float32) + b_ref[...]


def _phase_a(feat, w12, loop_weight, bias2d):
    br = 2000
    grid = N // br
    return pl.pallas_call(
        _a_body,
        grid=(grid,),
        in_specs=[
            pl.BlockSpec((br, F), lambda i: (i, 0)),
            pl.BlockSpec((KR, F, F), lambda i: (0, 0, 0)),
            pl.BlockSpec((F, F), lambda i: (0, 0)),
            pl.BlockSpec((1, F), lambda i: (0, 0)),
        ],
        out_specs=[
            pl.BlockSpec((br, KR * F), lambda i: (i, 0)),
            pl.BlockSpec((br, F), lambda i: (i, 0)),
        ],
        out_shape=[
            jax.ShapeDtypeStruct((N, KR * F), jnp.float32),
            jax.ShapeDtypeStruct((N, F), jnp.float32),
        ],
    )(feat, w12, loop_weight, bias2d)


_SPLAT_DNUMS = lax.GatherDimensionNumbers(
    offset_dims=(), collapsed_slice_dims=(0,), start_index_map=(0,))


def _splat(v, j):
    idx = jnp.full((16, 1), j, jnp.int32)
    return lax.gather(v, idx, _SPLAT_DNUMS, slice_sizes=(1,),
                      mode=lax.GatherScatterMode.PROMISE_IN_BOUNDS)


def _b_body(t_hbm, md_hbm, tru_hbm, zero_hbm, out_hbm,
            md_v, tru_v, idx_v, dst_v, rows_v, msg_v, agg,
            sem_m, sem_t, sem_g, sem_s):
    cid = lax.axis_index("c")
    sid = lax.axis_index("s")
    wid = sid * NC + cid

    pltpu.sync_copy(zero_hbm.at[pl.ds(sid * RPT, RPT)],
                    agg.at[pl.ds(sid * RPT, RPT)])
    plsc.subcore_barrier()

    row0 = wid * NCH

    def start_md(g, s4):
        pltpu.async_copy(md_hbm.at[pl.ds((row0 + g) * MDW, MDW)],
                         md_v.at[s4], sem_m[s4])
        pltpu.async_copy(tru_hbm.at[pl.ds((row0 + g) * TW, TW)],
                         tru_v.at[s4], sem_t[s4])

    def wait_md(s4):
        pltpu.make_async_copy(md_hbm.at[pl.ds(0, MDW)], md_v.at[s4],
                              sem_m[s4]).wait()
        pltpu.make_async_copy(tru_hbm.at[pl.ds(0, TW)], tru_v.at[s4],
                              sem_t[s4]).wait()

    def prep_gather(s4, b2):
        idx_v[b2, pl.ds(0, C)] = (md_v[s4, pl.ds(0, C)] * NRELS
                                  + md_v[s4, pl.ds(C, C)])
        dst_v[s4, pl.ds(0, C)] = md_v[s4, pl.ds(2 * C, C)]
        pltpu.async_copy(t_hbm.at[idx_v.at[b2]], rows_v.at[b2], sem_g[b2])

    def wait_gather(b2):
        pltpu.make_async_copy(t_hbm.at[idx_v.at[b2]], rows_v.at[b2],
                              sem_g[b2]).wait()

    def wait_scatter(b2):
        pltpu.make_async_copy(msg_v.at[b2], agg.at[dst_v.at[0]],
                              sem_s[b2]).wait()

    def compute_chunk(s4, b2):
        tq0 = tru_v[s4, pl.ds(0, 16)]
        tq1 = tru_v[s4, pl.ds(C, 16)]
        tq2 = tru_v[s4, pl.ds(2 * C, 16)]

        for j in range(C):
            t0 = _splat(tq0, j)
            t1 = _splat(tq1, j)
            t2 = _splat(tq2, j)
            for s in range(F // 16):
                a = rows_v[b2, j, pl.ds(s * 16, 16)] * t0
                a = a + rows_v[b2, j, pl.ds(F + s * 16, 16)] * t1
                a = a + rows_v[b2, j, pl.ds(2 * F + s * 16, 16)] * t2
                msg_v[b2, j, pl.ds(s * 16, 16)] = a
        pltpu.async_copy(msg_v.at[b2], agg.at[dst_v.at[s4]], sem_s[b2],
                         add=True)

    for g in range(4):
        start_md(g, g)
    for g in range(2):
        wait_md(g)
        prep_gather(g, g)

    def quad_body(i, carry):
        for b4 in range(4):
            g = i * 4 + b4
            b2 = b4 % 2
            wait_gather(b2)

            @pl.when(g >= 2)
            def _():
                wait_scatter(b2)

            compute_chunk(b4, b2)

            @pl.when(g + 2 < NCH)
            def _():
                wait_md((b4 + 2) % 4)
                prep_gather((b4 + 2) % 4, b2)

            @pl.when(g + 4 < NCH)
            def _():
                start_md(g + 4, b4)

        return carry

    lax.fori_loop(0, NCH // 4, quad_body, 0)
    wait_scatter(0)
    wait_scatter(1)

    plsc.subcore_barrier()
    pltpu.sync_copy(agg.at[pl.ds(sid * RPT, RPT)],
                    out_hbm.at[cid, pl.ds(sid * RPT, RPT)])


_phase_b = functools.partial(
    pl.kernel,
    out_type=jax.ShapeDtypeStruct((NC, N_PAD, F), jnp.float32),
    mesh=plsc.VectorSubcoreMesh(core_axis_name="c", subcore_axis_name="s"),
    scratch_types=[
        pltpu.VMEM((4, MDW), jnp.int32),
        pltpu.VMEM((4, TW), jnp.float32),
        pltpu.VMEM((2, C), jnp.int32),
        pltpu.VMEM((4, C), jnp.int32),
        pltpu.VMEM((2, C, RW), jnp.float32),
        pltpu.VMEM((2, C, F), jnp.float32),
        pltpu.VMEM_SHARED((N_PAD, F), jnp.float32),
        [pltpu.SemaphoreType.DMA] * 4,
        [pltpu.SemaphoreType.DMA] * 4,
        [pltpu.SemaphoreType.DMA] * 2,
        [pltpu.SemaphoreType.DMA] * 2,
    ],
)(_b_body)


def _c_body(p_ref, sl_ref, out_ref):
    out_ref[...] = p_ref[0] + p_ref[1] + sl_ref[...]


def _phase_c(partials, selfloop):
    br = 1000
    grid = N // br
    return pl.pallas_call(
        _c_body,
        grid=(grid,),
        in_specs=[
            pl.BlockSpec((NC, br, F), lambda i: (0, i, 0)),
            pl.BlockSpec((br, F), lambda i: (i, 0)),
        ],
        out_specs=pl.BlockSpec((br, F), lambda i: (i, 0)),
        out_shape=jax.ShapeDtypeStruct((N, F), jnp.float32),
    )(partials, selfloop)


def kernel(feat, edge_index, etypes, truth_value, loop_weight, weight, h_bias):
    w12 = weight.reshape(KR, F, F)
    bias2d = h_bias.reshape(1, F)
    g, selfloop = _phase_a(feat, w12, loop_weight, bias2d)
    table = g.reshape(N * NRELS, RW)

    pad = E_PAD - E
    src = jnp.concatenate(
        [edge_index[0], jnp.zeros((pad,), jnp.int32)]).reshape(NCHT, C)
    et = jnp.concatenate(
        [etypes, jnp.zeros((pad,), jnp.int32)]).reshape(NCHT, C)
    dst = jnp.concatenate(
        [edge_index[1], jnp.zeros((pad,), jnp.int32)]).reshape(NCHT, C)
    tru = jnp.concatenate(
        [truth_value.reshape(E, NRULES),
         jnp.zeros((pad, NRULES), jnp.float32)])
    tru = tru.reshape(NCHT, C, NRULES).transpose(0, 2, 1).reshape(-1)
    md = jnp.concatenate([src, et, dst], axis=1).reshape(-1)
    zero = jnp.zeros((N_PAD, F), jnp.float32)

    partials = _phase_b(table, md, tru, zero)
    return _phase_c(partials, selfloop)


# speedup vs baseline: 1.2282x; 1.0396x over previous
import functools

import jax
import jax.numpy as jnp
from jax import lax
from jax.experimental import pallas as pl
from jax.experimental.pallas import tpu as pltpu
from jax.experimental.pallas import tpu_sc as plsc

N = 10000
E = 160000
F = 128
NRELS = 4
NRULES = 3
KR = NRELS * NRULES
RW = NRULES * F

NC = 2
NS = 16
NW = NC * NS
C = 16
NCHT = 10240
E_PAD = NCHT * C
NCH = NCHT // NW
MDW = 128
N_PAD = 10112
RPT = N_PAD // NS


def _a_body(feat_ref, w_ref, lw_ref, b_ref, g_ref, sl_ref):
    x = feat_ref[...]
    xb = x.astype(jnp.bfloat16)
    for j in range(KR):
        g_ref[:, j * F:(j + 1) * F] = jnp.dot(
            xb, w_ref[j].astype(jnp.bfloat16),
            preferred_element_type=jnp.float32)
    sl_ref[...] = jnp.dot(x, lw_ref[...],
                          preferred_element_type=jnp.float32) + b_ref[...]


def _phase_a(feat, w12, loop_weight, bias2d):
    br = 2000
    grid = N // br
    return pl.pallas_call(
        _a_body,
        grid=(grid,),
        in_specs=[
            pl.BlockSpec((br, F), lambda i: (i, 0)),
            pl.BlockSpec((KR, F, F), lambda i: (0, 0, 0)),
            pl.BlockSpec((F, F), lambda i: (0, 0)),
            pl.BlockSpec((1, F), lambda i: (0, 0)),
        ],
        out_specs=[
            pl.BlockSpec((br, KR * F), lambda i: (i, 0)),
            pl.BlockSpec((br, F), lambda i: (i, 0)),
        ],
        out_shape=[
            jax.ShapeDtypeStruct((N, KR * F), jnp.float32),
            jax.ShapeDtypeStruct((N, F), jnp.float32),
        ],
    )(feat, w12, loop_weight, bias2d)


_SPLAT_DNUMS = lax.GatherDimensionNumbers(
    offset_dims=(), collapsed_slice_dims=(0,), start_index_map=(0,))


def _splat(v, j):
    idx = jnp.full((16, 1), j, jnp.int32)
    return lax.gather(v, idx, _SPLAT_DNUMS, slice_sizes=(1,),
                      mode=lax.GatherScatterMode.PROMISE_IN_BOUNDS)


def _b_body(t_hbm, md_hbm, zero_hbm, out_hbm,
            md_v, idx_v, dst_v, rows_v, msg_v, agg,
            sem_m, sem_g, sem_s):
    cid = lax.axis_index("c")
    sid = lax.axis_index("s")
    wid = sid * NC + cid

    pltpu.sync_copy(zero_hbm.at[pl.ds(sid * RPT, RPT)],
                    agg.at[pl.ds(sid * RPT, RPT)])
    plsc.subcore_barrier()

    row0 = wid * NCH

    def start_md(g, s4):
        pltpu.async_copy(md_hbm.at[pl.ds((row0 + g) * MDW, MDW)],
                         md_v.at[s4], sem_m[s4])

    def wait_md(s4):
        pltpu.make_async_copy(md_hbm.at[pl.ds(0, MDW)], md_v.at[s4],
                              sem_m[s4]).wait()

    def prep_gather(s4, b2):
        fidx = (md_v[s4, pl.ds(0, C)] * float(NRELS)
                + md_v[s4, pl.ds(C, C)])
        idx_v[b2, pl.ds(0, C)] = fidx.astype(jnp.int32)
        dst_v[s4, pl.ds(0, C)] = md_v[s4, pl.ds(2 * C, C)].astype(jnp.int32)
        pltpu.async_copy(t_hbm.at[idx_v.at[b2]], rows_v.at[b2], sem_g[b2])

    def wait_gather(b2):
        pltpu.make_async_copy(t_hbm.at[idx_v.at[b2]], rows_v.at[b2],
                              sem_g[b2]).wait()

    def wait_scatter(b2):
        pltpu.make_async_copy(msg_v.at[b2], agg.at[dst_v.at[0]],
                              sem_s[b2]).wait()

    def compute_chunk(s4, b2):
        tq0 = md_v[s4, pl.ds(3 * C, 16)]
        tq1 = md_v[s4, pl.ds(4 * C, 16)]
        tq2 = md_v[s4, pl.ds(5 * C, 16)]

        for j in range(C):
            t0 = _splat(tq0, j)
            t1 = _splat(tq1, j)
            t2 = _splat(tq2, j)
            for s in range(F // 16):
                a = rows_v[b2, j, pl.ds(s * 16, 16)] * t0
                a = a + rows_v[b2, j, pl.ds(F + s * 16, 16)] * t1
                a = a + rows_v[b2, j, pl.ds(2 * F + s * 16, 16)] * t2
                msg_v[b2, j, pl.ds(s * 16, 16)] = a
        pltpu.async_copy(msg_v.at[b2], agg.at[dst_v.at[s4]], sem_s[b2],
                         add=True)

    for g in range(4):
        start_md(g, g)
    for g in range(2):
        wait_md(g)
        prep_gather(g, g)

    def quad_body(i, carry):
        for b4 in range(4):
            g = i * 4 + b4
            b2 = b4 % 2
            wait_gather(b2)

            @pl.when(g >= 2)
            def _():
                wait_scatter(b2)

            compute_chunk(b4, b2)

            @pl.when(g + 2 < NCH)
            def _():
                wait_md((b4 + 2) % 4)
                prep_gather((b4 + 2) % 4, b2)

            @pl.when(g + 4 < NCH)
            def _():
                start_md(g + 4, b4)

        return carry

    lax.fori_loop(0, NCH // 4, quad_body, 0)
    wait_scatter(0)
    wait_scatter(1)

    plsc.subcore_barrier()
    pltpu.sync_copy(agg.at[pl.ds(sid * RPT, RPT)],
                    out_hbm.at[cid, pl.ds(sid * RPT, RPT)])


_phase_b = functools.partial(
    pl.kernel,
    out_type=jax.ShapeDtypeStruct((NC, N_PAD, F), jnp.float32),
    mesh=plsc.VectorSubcoreMesh(core_axis_name="c", subcore_axis_name="s"),
    scratch_types=[
        pltpu.VMEM((4, MDW), jnp.float32),
        pltpu.VMEM((2, C), jnp.int32),
        pltpu.VMEM((4, C), jnp.int32),
        pltpu.VMEM((2, C, RW), jnp.float32),
        pltpu.VMEM((2, C, F), jnp.float32),
        pltpu.VMEM_SHARED((N_PAD, F), jnp.float32),
        [pltpu.SemaphoreType.DMA] * 4,
        [pltpu.SemaphoreType.DMA] * 2,
        [pltpu.SemaphoreType.DMA] * 2,
    ],
)(_b_body)


def _c_body(p_ref, sl_ref, out_ref):
    out_ref[...] = p_ref[0] + p_ref[1] + sl_ref[...]


def _phase_c(partials, selfloop):
    br = 1000
    grid = N // br
    return pl.pallas_call(
        _c_body,
        grid=(grid,),
        in_specs=[
            pl.BlockSpec((NC, br, F), lambda i: (0, i, 0)),
            pl.BlockSpec((br, F), lambda i: (i, 0)),
        ],
        out_specs=pl.BlockSpec((br, F), lambda i: (i, 0)),
        out_shape=jax.ShapeDtypeStruct((N, F), jnp.float32),
    )(partials, selfloop)


def kernel(feat, edge_index, etypes, truth_value, loop_weight, weight, h_bias):
    w12 = weight.reshape(KR, F, F)
    bias2d = h_bias.reshape(1, F)
    g, selfloop = _phase_a(feat, w12, loop_weight, bias2d)
    table = g.reshape(N * NRELS, RW)

    pad = E_PAD - E
    src = jnp.concatenate(
        [edge_index[0], jnp.zeros((pad,), jnp.int32)]).reshape(NCHT, C)
    et = jnp.concatenate(
        [etypes, jnp.zeros((pad,), jnp.int32)]).reshape(NCHT, C)
    dst = jnp.concatenate(
        [edge_index[1], jnp.zeros((pad,), jnp.int32)]).reshape(NCHT, C)
    tru = jnp.concatenate(
        [truth_value.reshape(E, NRULES),
         jnp.zeros((pad, NRULES), jnp.float32)])
    tru = tru.reshape(NCHT, C, NRULES).transpose(0, 2, 1).reshape(NCHT, 3 * C)
    md = jnp.concatenate(
        [src.astype(jnp.float32), et.astype(jnp.float32),
         dst.astype(jnp.float32), tru,
         jnp.zeros((NCHT, MDW - 6 * C), jnp.float32)], axis=1).reshape(-1)
    zero = jnp.zeros((N_PAD, F), jnp.float32)

    partials = _phase_b(table, md, zero)
    return _phase_c(partials, selfloop)
